# baseline (speedup 1.0000x reference)
import functools

import jax
import jax.numpy as jnp
from jax import lax
from jax.experimental import pallas as pl
from jax.experimental.pallas import tpu as pltpu
from jax.experimental.pallas import tpu_sc as plsc

N, C, H, W = 5000, 4, 56, 56
L = 16
NW = 32
BPW = N // NW
REM = N - NW * BPW
QMAX = BPW + 1
WIN = 168
PADL = 5008

_mesh = plsc.VectorSubcoreMesh(core_axis_name="c", subcore_axis_name="s")


@functools.partial(
    pl.kernel,
    mesh=_mesh,
    compiler_params=pltpu.CompilerParams(needs_layout_passes=False),
    out_type=jax.ShapeDtypeStruct((N, 1, H, W), jnp.float32),
    scratch_types=[
        pltpu.VMEM((WIN,), jnp.int32),
        pltpu.VMEM((BPW + L,), jnp.int32),
        pltpu.SemaphoreType.DMA,
    ],
)
def _gather_rows(table_hbm, labels_hbm, out_hbm, win_v, lbl_v, sem):
    wid = lax.axis_index("s") * 2 + lax.axis_index("c")
    has_extra = wid < REM
    start = wid * BPW + jnp.minimum(wid, REM)
    win_start = (start // 8) * 8
    off0 = start - win_start
    pltpu.sync_copy(labels_hbm.at[pl.ds(win_start, WIN)], win_v)
    for j in range((BPW + L) // L):
        lbl_v[pl.ds(j * L, L)] = win_v[pl.ds(off0 + j * L, L)]

    iota = lax.iota(jnp.int32, L)

    def fire(r):
        j, l = divmod(r, L)
        vec = lbl_v[pl.ds(j * L, L)]
        lbl_r = jnp.sum(jnp.where(iota == l, vec, 0))
        row = (start + r) * C + lbl_r
        return pltpu.async_copy(table_hbm.at[row], out_hbm.at[start + r, 0],
                                sem)

    copies = []
    for r in range(BPW):
        copies.append(fire(r))

    @pl.when(has_extra)
    def _():
        fire(BPW).wait()

    for cp in copies:
        cp.wait()


def kernel(x, labels):
    table = x.reshape(N * C, H, W)
    lbl = jnp.pad(labels.astype(jnp.int32), (0, PADL - N))
    return _gather_rows(table, lbl)


# speedup vs baseline: 6.9008x; 6.9008x over previous
import functools

import jax
import jax.numpy as jnp
from jax import lax
from jax.experimental import pallas as pl
from jax.experimental.pallas import tpu as pltpu
from jax.experimental.pallas import tpu_sc as plsc

N, C, H, W = 5000, 4, 56, 56
L = 16
NW = 32
BPW = N // NW
REM = N - NW * BPW
WIN = 184
PADL = 5024
K = 8
NCH = BPW // K
TAIL = BPW - NCH * K

_mesh = plsc.VectorSubcoreMesh(core_axis_name="c", subcore_axis_name="s")


@functools.partial(
    pl.kernel,
    mesh=_mesh,
    compiler_params=pltpu.CompilerParams(needs_layout_passes=False),
    out_type=jax.ShapeDtypeStruct((N, 1, H, W), jnp.float32),
    scratch_types=[
        pltpu.VMEM((WIN,), jnp.int32),
        pltpu.VMEM((K, H, W), jnp.float32),
        pltpu.VMEM((K, H, W), jnp.float32),
        pltpu.SemaphoreType.DMA,
        pltpu.SemaphoreType.DMA,
        pltpu.SemaphoreType.DMA,
        pltpu.SemaphoreType.DMA,
    ],
)
def _gather_rows(x_hbm, labels_hbm, out_hbm, win_v, buf0, buf1,
                 gsem0, gsem1, wsem0, wsem1):
    wid = lax.axis_index("s") * 2 + lax.axis_index("c")
    has_extra = wid < REM
    start = wid * BPW + jnp.minimum(wid, REM)
    win_start = (start // 8) * 8
    off0 = start - win_start
    pltpu.sync_copy(labels_hbm.at[pl.ds(win_start, WIN)], win_v)

    bufs = (buf0, buf1)
    gsems = (gsem0, gsem1)
    wsems = (wsem0, wsem1)

    def fire(local_row, buf, slot, gsem):
        lbl = win_v[pl.ds(off0 + local_row, L)][0]
        return pltpu.async_copy(
            x_hbm.at[start + local_row, lbl], buf.at[slot], gsem)

    pending_write = [None, None]
    for g in range(NCH + 1):
        s = g % 2
        nrows = K if g < NCH else TAIL
        if pending_write[s] is not None:
            pending_write[s].wait()
            pending_write[s] = None
        copies = [fire(g * K + r, bufs[s], r, gsems[s]) for r in range(nrows)]
        for cp in copies:
            cp.wait()
        pending_write[s] = pltpu.async_copy(
            bufs[s].at[pl.ds(0, nrows)],
            out_hbm.at[pl.ds(start + g * K, nrows), 0], wsems[s])

    @pl.when(has_extra)
    def _():
        r = BPW
        lbl = win_v[pl.ds(off0 + r, L)][0]
        pltpu.async_copy(x_hbm.at[start + r, lbl],
                         out_hbm.at[start + r, 0], gsems[0]).wait()

    for s in range(2):
        if pending_write[s] is not None:
            pending_write[s].wait()


def kernel(x, labels):
    lbl = jnp.pad(labels.astype(jnp.int32), (0, PADL - N))
    return _gather_rows(x, lbl)


# speedup vs baseline: 20.3984x; 2.9559x over previous
import functools

import jax
import jax.numpy as jnp
from jax import lax
from jax.experimental import pallas as pl
from jax.experimental.pallas import tpu as pltpu
from jax.experimental.pallas import tpu_sc as plsc

N, C, H, W = 5000, 4, 56, 56
HW = H * W
NPAD = 5120
NT = 8
NMAIN = N - NT
L = 16
NW = 32
HB = 8
NOFF = (0, 1280, 2560, 3840)
NBQ = (1280, 1280, 1280, 1152)
NB = NBQ[0]
NBATCH = HW // (NW * HB)
XTRA = HW // HB - NW * NBATCH

_mesh = plsc.VectorSubcoreMesh(core_axis_name="c", subcore_axis_name="s")


@functools.partial(
    pl.kernel,
    mesh=_mesh,
    compiler_params=pltpu.CompilerParams(needs_layout_passes=False),
    out_type=jax.ShapeDtypeStruct((HW, NPAD), jnp.float32),
    scratch_types=[
        pltpu.VMEM((NPAD,), jnp.int32),
        pltpu.VMEM((C, HB, NB), jnp.float32),
        pltpu.VMEM((C, HB, NB), jnp.float32),
        pltpu.VMEM((HB, NB), jnp.float32),
        pltpu.VMEM((HB, NB), jnp.float32),
        pltpu.VMEM((C, HB, NT), jnp.float32),
        pltpu.VMEM((HB, 128), jnp.float32),
        pltpu.SemaphoreType.DMA,
        pltpu.SemaphoreType.DMA,
        pltpu.SemaphoreType.DMA,
        pltpu.SemaphoreType.DMA,
        pltpu.SemaphoreType.DMA,
    ],
)
def _select_kernel(xt_hbm, xtail_hbm, labels_hbm, out_hbm, lbl_v,
                   ibuf0, ibuf1, obuf0, obuf1, ibuf_t, obuf_t,
                   g0, g1, w0, w1, gt):
    wid = lax.axis_index("s") * 2 + lax.axis_index("c")
    hw_start = wid * (NBATCH * HB) + HB * jnp.minimum(wid, XTRA)
    nb = NBATCH + jnp.where(wid < XTRA, 1, 0)
    pltpu.sync_copy(labels_hbm, lbl_v)
    iota = lax.iota(jnp.int32, L)
    tail_mask = iota < NT
    tail_idx = jnp.minimum(iota, NT - 1)

    ibufs = (ibuf0, ibuf1)
    obufs = (obuf0, obuf1)
    gsems = (g0, g1)
    wsems = (w0, w1)

    def compute(ibuf, obuf, n0, ngroups):
        def kbody(k, _):
            lbl16 = lbl_v[pl.ds(n0 + k * L, L)]
            n_idx = k * L + iota
            for h in range(HB):
                h_idx = jnp.full((L,), h, jnp.int32)
                v = plsc.load_gather(ibuf, [lbl16, h_idx, n_idx])
                plsc.store_scatter(obuf, [h_idx, n_idx], v)
            return 0
        lax.fori_loop(0, ngroups, kbody, 0, unroll=2)

    def drain(s, size):
        pltpu.make_async_copy(
            out_hbm.at[pl.ds(0, HB), pl.ds(0, size)],
            obufs[s].at[:, pl.ds(0, size)], wsems[s]).wait()

    def batch_body(b, _):
        hw0 = hw_start + b * HB

        def fire_in(q):
            s = q % 2
            return pltpu.async_copy(
                xt_hbm.at[:, pl.ds(hw0, HB), pl.ds(NOFF[q], NBQ[q])],
                ibufs[s].at[:, :, pl.ds(0, NBQ[q])], gsems[s])

        pending = fire_in(0)
        for q in range(4):
            s = q % 2
            nxt = fire_in(q + 1) if q + 1 < 4 else None
            pending.wait()
            if q < 2:
                @pl.when(b > 0)
                def _():
                    drain(s, NBQ[q + 2])
            else:
                drain(s, NBQ[q - 2])
            compute(ibufs[s], obufs[s], NOFF[q], NBQ[q] // L)
            pltpu.async_copy(
                obufs[s].at[:, pl.ds(0, NBQ[q])],
                out_hbm.at[pl.ds(hw0, HB), pl.ds(NOFF[q], NBQ[q])],
                wsems[s])
            pending = nxt

        pltpu.async_copy(xtail_hbm.at[:, pl.ds(hw0, HB)], ibuf_t, gt).wait()

        def tbody(h):
            h_idx = jnp.full((L,), h, jnp.int32)
            lbl16 = lbl_v[pl.ds(NMAIN, L)]
            v = plsc.load_gather(ibuf_t, [lbl16, h_idx, tail_idx],
                                 mask=tail_mask)
            plsc.store_scatter(obuf_t, [h_idx, tail_idx], v, mask=tail_mask)
        for h in range(HB):
            tbody(h)
        pltpu.sync_copy(obuf_t,
                        out_hbm.at[pl.ds(hw0, HB), pl.ds(NMAIN, 128)])
        return 0

    lax.fori_loop(0, nb, batch_body, 0)

    drain(0, NBQ[2])
    drain(1, NBQ[3])


def kernel(x, labels):
    xt = jnp.transpose(x, (1, 2, 3, 0)).reshape(C, HW, N)
    xtail = jnp.transpose(x[NMAIN:], (1, 2, 3, 0)).reshape(C, HW, NT)
    lbl = jnp.pad(labels.astype(jnp.int32), (0, NPAD - N))
    out2 = _select_kernel(xt, xtail, lbl)
    return jnp.transpose(out2[:, :N].reshape(H, W, N)[None], (3, 0, 1, 2))


# speedup vs baseline: 25.2748x; 1.2391x over previous
import functools

import jax
import jax.numpy as jnp
from jax import lax
from jax.experimental import pallas as pl
from jax.experimental.pallas import tpu as pltpu
from jax.experimental.pallas import tpu_sc as plsc

N, C, H, W = 5000, 4, 56, 56
HW = H * W
NPAD = 5120
NT = 8
NMAIN = N - NT
L = 16
NW = 32
HB = 8
NOFF = (0, 640, 1280, 1920, 2560, 3200, 3840, 4480)
NBQ = (640, 640, 640, 640, 640, 640, 640, 512)
NB = NBQ[0]
NQ = len(NBQ)
NBATCH = HW // (NW * HB)
XTRA = HW // HB - NW * NBATCH
HMAX = (NBATCH + 1) * HB

_mesh = plsc.VectorSubcoreMesh(core_axis_name="c", subcore_axis_name="s")


@functools.partial(
    pl.kernel,
    mesh=_mesh,
    compiler_params=pltpu.CompilerParams(needs_layout_passes=False),
    out_type=jax.ShapeDtypeStruct((HW, NPAD), jnp.float32),
    scratch_types=[
        pltpu.VMEM((NPAD,), jnp.int32),
        pltpu.VMEM((C, HB, NB), jnp.float32),
        pltpu.VMEM((C, HB, NB), jnp.float32),
        pltpu.VMEM((HB, NB), jnp.float32),
        pltpu.VMEM((HB, NB), jnp.float32),
        pltpu.VMEM((C, HMAX, NT), jnp.float32),
        pltpu.VMEM((HB, 128), jnp.float32),
        pltpu.VMEM((HB, 128), jnp.float32),
        pltpu.SemaphoreType.DMA,
        pltpu.SemaphoreType.DMA,
        pltpu.SemaphoreType.DMA,
        pltpu.SemaphoreType.DMA,
        pltpu.SemaphoreType.DMA,
        pltpu.SemaphoreType.DMA,
    ],
)
def _select_kernel(xt_hbm, xtail_hbm, labels_hbm, out_hbm, lbl_v,
                   ibuf0, ibuf1, obuf0, obuf1, ibuf_t, obuf_t0, obuf_t1,
                   g0, g1, w0, w1, gt, wt):
    wid = lax.axis_index("s") * 2 + lax.axis_index("c")
    has_extra = wid < XTRA
    hw_start = wid * (NBATCH * HB) + HB * jnp.minimum(wid, XTRA)
    nb = NBATCH + jnp.where(has_extra, 1, 0)
    pltpu.sync_copy(labels_hbm, lbl_v)
    iota = lax.iota(jnp.int32, L)
    tail_mask = iota < NT
    tail_idx = jnp.minimum(iota, NT - 1)

    ibufs = (ibuf0, ibuf1)
    obufs = (obuf0, obuf1)
    gsems = (g0, g1)
    wsems = (w0, w1)

    def compute(ibuf, obuf, n0, ngroups):
        def kbody(k, _):
            lbl16 = lbl_v[pl.ds(n0 + k * L, L)]
            n_idx = k * L + iota
            for h in range(HB):
                h_idx = jnp.full((L,), h, jnp.int32)
                v = plsc.load_gather(ibuf, [lbl16, h_idx, n_idx])
                plsc.store_scatter(obuf, [h_idx, n_idx], v)
            return 0
        lax.fori_loop(0, ngroups, kbody, 0, unroll=4)

    def fire_in(hw0, q):
        return pltpu.async_copy(
            xt_hbm.at[:, pl.ds(hw0, HB), pl.ds(NOFF[q], NBQ[q])],
            ibufs[q % 2].at[:, :, pl.ds(0, NBQ[q])], gsems[q % 2])

    def drain_in0():
        pltpu.make_async_copy(
            xt_hbm.at[:, pl.ds(0, HB), pl.ds(0, NB)],
            ibufs[0], gsems[0]).wait()

    def drain_out(s, size):
        pltpu.make_async_copy(
            out_hbm.at[pl.ds(0, HB), pl.ds(0, size)],
            obufs[s].at[:, pl.ds(0, size)], wsems[s]).wait()

    def fire_out(hw0, q):
        pltpu.async_copy(
            obufs[q % 2].at[:, pl.ds(0, NBQ[q])],
            out_hbm.at[pl.ds(hw0, HB), pl.ds(NOFF[q], NBQ[q])],
            wsems[q % 2])

    def run_block(hw0, q, reclaim_size, reclaim_pred=None):
        s = q % 2
        if reclaim_pred is None:
            drain_out(s, reclaim_size)
        else:
            @pl.when(reclaim_pred)
            def _():
                drain_out(s, reclaim_size)
        compute(ibufs[s], obufs[s], NOFF[q], NBQ[q] // L)
        fire_out(hw0, q)

    fire_in(hw_start, 0)

    def batch_body(b, _):
        hw0 = hw_start + b * HB
        pending = fire_in(hw0, 1)
        drain_in0()
        run_block(hw0, 0, NBQ[NQ - 2], b > 0)
        for q in range(1, NQ):
            nxt = fire_in(hw0, q + 1) if q + 1 < NQ else None
            if q + 1 == NQ:
                @pl.when(b + 1 < nb)
                def _():
                    fire_in(hw0 + HB, 0)
            pending.wait()
            if q == 1:
                run_block(hw0, q, NBQ[NQ - 1], b > 0)
            else:
                run_block(hw0, q, NBQ[q - 2])
            pending = nxt
        return 0

    lax.fori_loop(0, nb, batch_body, 0)
    drain_out(0, NBQ[NQ - 2])
    drain_out(1, NBQ[NQ - 1])

    obuf_ts = (obuf_t0, obuf_t1)
    wsem_ts = (wt, gt)
    lbl_tail = lbl_v[pl.ds(NMAIN, L)]

    def drain_tail(s):
        pltpu.make_async_copy(
            out_hbm.at[pl.ds(0, HB), pl.ds(0, 128)], obuf_ts[s],
            wsem_ts[s]).wait()

    def tail(nrows):
        pltpu.async_copy(
            xtail_hbm.at[:, pl.ds(hw_start, nrows)],
            ibuf_t.at[:, pl.ds(0, nrows)], gt).wait()
        nbt = nrows // HB
        for i in range(nbt):
            if i >= 2:
                drain_tail(i % 2)
            ob = obuf_ts[i % 2]
            for j in range(HB):
                h_idx = jnp.full((L,), i * HB + j, jnp.int32)
                v = plsc.load_gather(ibuf_t, [lbl_tail, h_idx, tail_idx],
                                     mask=tail_mask)
                plsc.store_scatter(ob, [jnp.full((L,), j, jnp.int32),
                                        tail_idx], v, mask=tail_mask)
            pltpu.async_copy(
                ob, out_hbm.at[pl.ds(hw_start + i * HB, HB),
                               pl.ds(NMAIN, 128)], wsem_ts[i % 2])
        for s in range(2):
            drain_tail((nbt + s) % 2)

    @pl.when(has_extra)
    def _():
        tail(HMAX)

    @pl.when(jnp.logical_not(has_extra))
    def _():
        tail(NBATCH * HB)


def kernel(x, labels):
    xt = jnp.transpose(x, (1, 2, 3, 0)).reshape(C, HW, N)
    xtail = jnp.transpose(x[NMAIN:], (1, 2, 3, 0)).reshape(C, HW, NT)
    lbl = jnp.pad(labels.astype(jnp.int32), (0, NPAD - N))
    out2 = _select_kernel(xt, xtail, lbl)
    return jnp.transpose(out2[:, :N].reshape(H, W, N)[None], (3, 0, 1, 2))


# speedup vs baseline: 25.3460x; 1.0028x over previous
import functools

import jax
import jax.numpy as jnp
from jax import lax
from jax.experimental import pallas as pl
from jax.experimental.pallas import tpu as pltpu
from jax.experimental.pallas import tpu_sc as plsc

N, C, H, W = 5000, 4, 56, 56
HW = H * W
NPAD = 5120
NT = 8
NMAIN = N - NT
L = 16
NW = 32
HB = 8
NOFF = (0, 640, 1280, 1920, 2560, 3200, 3840, 4480)
NBQ = (640, 640, 640, 640, 640, 640, 640, 512)
NB = NBQ[0]
NQ = len(NBQ)
NBATCH = HW // (NW * HB)
XTRA = HW // HB - NW * NBATCH
HMAX = (NBATCH + 1) * HB

_mesh = plsc.VectorSubcoreMesh(core_axis_name="c", subcore_axis_name="s")


@functools.partial(
    pl.kernel,
    mesh=_mesh,
    compiler_params=pltpu.CompilerParams(needs_layout_passes=False),
    out_type=jax.ShapeDtypeStruct((HW, NPAD), jnp.float32),
    scratch_types=[
        pltpu.VMEM((NPAD,), jnp.int32),
        pltpu.VMEM((C, HB, NB), jnp.float32),
        pltpu.VMEM((C, HB, NB), jnp.float32),
        pltpu.VMEM((HB, NB), jnp.float32),
        pltpu.VMEM((HB, NB), jnp.float32),
        pltpu.VMEM((HB, NB), jnp.float32),
        pltpu.VMEM((HB, NB), jnp.float32),
        pltpu.VMEM((C, HMAX, NT), jnp.float32),
        pltpu.VMEM((HB, 128), jnp.float32),
        pltpu.VMEM((HB, 128), jnp.float32),
        pltpu.SemaphoreType.DMA,
        pltpu.SemaphoreType.DMA,
        pltpu.SemaphoreType.DMA,
        pltpu.SemaphoreType.DMA,
        pltpu.SemaphoreType.DMA,
        pltpu.SemaphoreType.DMA,
        pltpu.SemaphoreType.DMA,
        pltpu.SemaphoreType.DMA,
    ],
)
def _select_kernel(xt_hbm, xtail_hbm, labels_hbm, out_hbm, lbl_v,
                   ibuf0, ibuf1, obuf0, obuf1, obuf2, obuf3,
                   ibuf_t, obuf_t0, obuf_t1,
                   g0, g1, w0, w1, w2, w3, gt, wt):
    wid = lax.axis_index("s") * 2 + lax.axis_index("c")
    has_extra = wid < XTRA
    hw_start = wid * (NBATCH * HB) + HB * jnp.minimum(wid, XTRA)
    nb = NBATCH + jnp.where(has_extra, 1, 0)
    pltpu.sync_copy(labels_hbm, lbl_v)
    iota = lax.iota(jnp.int32, L)
    tail_mask = iota < NT
    tail_idx = jnp.minimum(iota, NT - 1)

    ibufs = (ibuf0, ibuf1)
    obufs = (obuf0, obuf1, obuf2, obuf3)
    gsems = (g0, g1)
    wsems = (w0, w1, w2, w3)

    def compute(ibuf, obuf, n0, ngroups):
        def kbody(k, _):
            lbl16 = lbl_v[pl.ds(n0 + k * L, L)]
            n_idx = k * L + iota
            for h in range(HB):
                h_idx = jnp.full((L,), h, jnp.int32)
                v = plsc.load_gather(ibuf, [lbl16, h_idx, n_idx])
                plsc.store_scatter(obuf, [h_idx, n_idx], v)
            return 0
        lax.fori_loop(0, ngroups, kbody, 0, unroll=4)

    def fire_in(hw0, q):
        return pltpu.async_copy(
            xt_hbm.at[:, pl.ds(hw0, HB), pl.ds(NOFF[q], NBQ[q])],
            ibufs[q % 2].at[:, :, pl.ds(0, NBQ[q])], gsems[q % 2])

    def drain_in0():
        pltpu.make_async_copy(
            xt_hbm.at[:, pl.ds(0, HB), pl.ds(0, NB)],
            ibufs[0], gsems[0]).wait()

    def drain_out(s, size):
        pltpu.make_async_copy(
            out_hbm.at[pl.ds(0, HB), pl.ds(0, size)],
            obufs[s].at[:, pl.ds(0, size)], wsems[s]).wait()

    def fire_out(hw0, q):
        pltpu.async_copy(
            obufs[q % 4].at[:, pl.ds(0, NBQ[q])],
            out_hbm.at[pl.ds(hw0, HB), pl.ds(NOFF[q], NBQ[q])],
            wsems[q % 4])

    def run_block(hw0, q, reclaim_size, reclaim_pred=None):
        s = q % 4
        if reclaim_pred is None:
            drain_out(s, reclaim_size)
        else:
            @pl.when(reclaim_pred)
            def _():
                drain_out(s, reclaim_size)
        compute(ibufs[q % 2], obufs[s], NOFF[q], NBQ[q] // L)
        fire_out(hw0, q)

    fire_in(hw_start, 0)

    def batch_body(b, _):
        hw0 = hw_start + b * HB
        pending = fire_in(hw0, 1)
        drain_in0()
        run_block(hw0, 0, NBQ[NQ - 4], b > 0)
        for q in range(1, NQ):
            nxt = fire_in(hw0, q + 1) if q + 1 < NQ else None
            if q + 1 == NQ:
                @pl.when(b + 1 < nb)
                def _():
                    fire_in(hw0 + HB, 0)
            pending.wait()
            if q < 4:
                run_block(hw0, q, NBQ[NQ - 4 + q], b > 0)
            else:
                run_block(hw0, q, NBQ[q - 4])
            pending = nxt
        return 0

    lax.fori_loop(0, nb, batch_body, 0)
    for s in range(4):
        drain_out(s, NBQ[NQ - 4 + s])

    obuf_ts = (obuf_t0, obuf_t1)
    wsem_ts = (wt, gt)
    lbl_tail = lbl_v[pl.ds(NMAIN, L)]

    def drain_tail(s):
        pltpu.make_async_copy(
            out_hbm.at[pl.ds(0, HB), pl.ds(0, 128)], obuf_ts[s],
            wsem_ts[s]).wait()

    def tail(nrows):
        pltpu.async_copy(
            xtail_hbm.at[:, pl.ds(hw_start, nrows)],
            ibuf_t.at[:, pl.ds(0, nrows)], gt).wait()
        nbt = nrows // HB
        for i in range(nbt):
            if i >= 2:
                drain_tail(i % 2)
            ob = obuf_ts[i % 2]
            for j in range(HB):
                h_idx = jnp.full((L,), i * HB + j, jnp.int32)
                v = plsc.load_gather(ibuf_t, [lbl_tail, h_idx, tail_idx],
                                     mask=tail_mask)
                plsc.store_scatter(ob, [jnp.full((L,), j, jnp.int32),
                                        tail_idx], v, mask=tail_mask)
            pltpu.async_copy(
                ob, out_hbm.at[pl.ds(hw_start + i * HB, HB),
                               pl.ds(NMAIN, 128)], wsem_ts[i % 2])
        for s in range(2):
            drain_tail((nbt + s) % 2)

    @pl.when(has_extra)
    def _():
        tail(HMAX)

    @pl.when(jnp.logical_not(has_extra))
    def _():
        tail(NBATCH * HB)


def kernel(x, labels):
    xt = jnp.transpose(x, (1, 2, 3, 0)).reshape(C, HW, N)
    xtail = jnp.transpose(x[NMAIN:], (1, 2, 3, 0)).reshape(C, HW, NT)
    lbl = jnp.pad(labels.astype(jnp.int32), (0, NPAD - N))
    out2 = _select_kernel(xt, xtail, lbl)
    return jnp.transpose(out2[:, :N].reshape(H, W, N)[None], (3, 0, 1, 2))


# speedup vs baseline: 29.9211x; 1.1805x over previous
import functools

import jax
import jax.numpy as jnp
from jax import lax
from jax.experimental import pallas as pl
from jax.experimental.pallas import tpu as pltpu
from jax.experimental.pallas import tpu_sc as plsc

N, C, H, W = 5000, 4, 56, 56
HW = H * W
NPAD = 5120
NT = 8
NMAIN = N - NT
L = 16
NW = 32
HB = 8
NOFF = (0, 640, 1280, 1920, 2560, 3200, 3840, 4480)
NBQ = (640, 640, 640, 640, 640, 640, 640, 512)
NB = NBQ[0]
NQ = len(NBQ)
NBATCH = HW // (NW * HB)
XTRA = HW // HB - NW * NBATCH
HMAX = (NBATCH + 1) * HB

_mesh = plsc.VectorSubcoreMesh(core_axis_name="c", subcore_axis_name="s")


@functools.partial(
    pl.kernel,
    mesh=_mesh,
    compiler_params=pltpu.CompilerParams(needs_layout_passes=False),
    out_type=jax.ShapeDtypeStruct((HW, NPAD), jnp.float32),
    scratch_types=[
        pltpu.VMEM((NPAD,), jnp.int32),
        pltpu.VMEM((C, HB, NB), jnp.float32),
        pltpu.VMEM((C, HB, NB), jnp.float32),
        pltpu.VMEM((HB, NB), jnp.float32),
        pltpu.VMEM((HB, NB), jnp.float32),
        pltpu.VMEM((HB, NB), jnp.float32),
        pltpu.VMEM((HB, NB), jnp.float32),
        pltpu.VMEM((C, HMAX, NT), jnp.float32),
        pltpu.VMEM((HB, 128), jnp.float32),
        pltpu.VMEM((HB, 128), jnp.float32),
        pltpu.SemaphoreType.DMA,
        pltpu.SemaphoreType.DMA,
        pltpu.SemaphoreType.DMA,
        pltpu.SemaphoreType.DMA,
        pltpu.SemaphoreType.DMA,
        pltpu.SemaphoreType.DMA,
        pltpu.SemaphoreType.DMA,
        pltpu.SemaphoreType.DMA,
    ],
)
def _select_kernel(xt_hbm, xtail_hbm, labels_hbm, out_hbm, lbl_v,
                   ibuf0, ibuf1, obuf0, obuf1, obuf2, obuf3,
                   ibuf_t, obuf_t0, obuf_t1,
                   g0, g1, w0, w1, w2, w3, gt, wt):
    wid = lax.axis_index("s") * 2 + lax.axis_index("c")
    has_extra = wid < XTRA
    hw_start = wid * (NBATCH * HB) + HB * jnp.minimum(wid, XTRA)
    nb = NBATCH + jnp.where(has_extra, 1, 0)
    pltpu.sync_copy(labels_hbm, lbl_v)
    iota = lax.iota(jnp.int32, L)
    tail_mask = iota < NT
    tail_idx = jnp.minimum(iota, NT - 1)

    ibufs = (ibuf0, ibuf1)
    obufs = (obuf0, obuf1, obuf2, obuf3)
    gsems = (g0, g1)
    wsems = (w0, w1, w2, w3)

    def compute(ibuf, obuf, n0, ngroups):
        def kbody(k, _):
            lbl16 = lbl_v[pl.ds(n0 + k * L, L)]
            n_idx = k * L + iota
            vs = [
                plsc.load_gather(
                    ibuf, [lbl16, jnp.full((L,), h, jnp.int32), n_idx])
                for h in range(HB)
            ]
            for h in range(HB):
                plsc.store_scatter(
                    obuf, [jnp.full((L,), h, jnp.int32), n_idx], vs[h])
            return 0
        lax.fori_loop(0, ngroups, kbody, 0, unroll=4)

    def fire_in(hw0, q):
        return pltpu.async_copy(
            xt_hbm.at[:, pl.ds(hw0, HB), pl.ds(NOFF[q], NBQ[q])],
            ibufs[q % 2].at[:, :, pl.ds(0, NBQ[q])], gsems[q % 2])

    def drain_in0():
        pltpu.make_async_copy(
            xt_hbm.at[:, pl.ds(0, HB), pl.ds(0, NB)],
            ibufs[0], gsems[0]).wait()

    def drain_out(s, size):
        pltpu.make_async_copy(
            out_hbm.at[pl.ds(0, HB), pl.ds(0, size)],
            obufs[s].at[:, pl.ds(0, size)], wsems[s]).wait()

    def fire_out(hw0, q):
        pltpu.async_copy(
            obufs[q % 4].at[:, pl.ds(0, NBQ[q])],
            out_hbm.at[pl.ds(hw0, HB), pl.ds(NOFF[q], NBQ[q])],
            wsems[q % 4])

    def run_block(hw0, q, reclaim_size, reclaim_pred=None):
        s = q % 4
        if reclaim_pred is None:
            drain_out(s, reclaim_size)
        else:
            @pl.when(reclaim_pred)
            def _():
                drain_out(s, reclaim_size)
        compute(ibufs[q % 2], obufs[s], NOFF[q], NBQ[q] // L)
        fire_out(hw0, q)

    fire_in(hw_start, 0)

    def batch_body(b, _):
        hw0 = hw_start + b * HB
        pending = fire_in(hw0, 1)
        drain_in0()
        run_block(hw0, 0, NBQ[NQ - 4], b > 0)
        for q in range(1, NQ):
            nxt = fire_in(hw0, q + 1) if q + 1 < NQ else None
            if q + 1 == NQ:
                @pl.when(b + 1 < nb)
                def _():
                    fire_in(hw0 + HB, 0)
            pending.wait()
            if q < 4:
                run_block(hw0, q, NBQ[NQ - 4 + q], b > 0)
            else:
                run_block(hw0, q, NBQ[q - 4])
            pending = nxt
        return 0

    lax.fori_loop(0, nb, batch_body, 0)
    for s in range(4):
        drain_out(s, NBQ[NQ - 4 + s])

    obuf_ts = (obuf_t0, obuf_t1)
    wsem_ts = (wt, gt)
    lbl_tail = lbl_v[pl.ds(NMAIN, L)]

    def drain_tail(s):
        pltpu.make_async_copy(
            out_hbm.at[pl.ds(0, HB), pl.ds(0, 128)], obuf_ts[s],
            wsem_ts[s]).wait()

    def tail(nrows):
        pltpu.async_copy(
            xtail_hbm.at[:, pl.ds(hw_start, nrows)],
            ibuf_t.at[:, pl.ds(0, nrows)], gt).wait()
        nbt = nrows // HB
        for i in range(nbt):
            if i >= 2:
                drain_tail(i % 2)
            ob = obuf_ts[i % 2]
            for j in range(HB):
                h_idx = jnp.full((L,), i * HB + j, jnp.int32)
                v = plsc.load_gather(ibuf_t, [lbl_tail, h_idx, tail_idx],
                                     mask=tail_mask)
                plsc.store_scatter(ob, [jnp.full((L,), j, jnp.int32),
                                        tail_idx], v, mask=tail_mask)
            pltpu.async_copy(
                ob, out_hbm.at[pl.ds(hw_start + i * HB, HB),
                               pl.ds(NMAIN, 128)], wsem_ts[i % 2])
        for s in range(2):
            drain_tail((nbt + s) % 2)

    @pl.when(has_extra)
    def _():
        tail(HMAX)

    @pl.when(jnp.logical_not(has_extra))
    def _():
        tail(NBATCH * HB)


def kernel(x, labels):
    xt = jnp.transpose(x, (1, 2, 3, 0)).reshape(C, HW, N)
    xtail = jnp.transpose(x[NMAIN:], (1, 2, 3, 0)).reshape(C, HW, NT)
    lbl = jnp.pad(labels.astype(jnp.int32), (0, NPAD - N))
    out2 = _select_kernel(xt, xtail, lbl)
    return jnp.transpose(out2[:, :N].reshape(H, W, N)[None], (3, 0, 1, 2))
